# baseline (device time: 74678 ns/iter reference)
import jax
import jax.numpy as jnp
from jax import lax
from jax.experimental import pallas as pl
from jax.experimental.pallas import tpu as pltpu

N_DEV = 32
E_PER = 2
N_EXP = 64
T_LOC = 256
D = 128
H = 256
CAP = 102


def kernel(x, router_W, route_idx, expert_W):
    del router_W
    x = x.astype(jnp.bfloat16)
    ew = expert_W.astype(jnp.bfloat16)
    ridx_row = route_idx.reshape(1, T_LOC)
    ridx_col = route_idx.reshape(T_LOC, 1)

    def body(x_ref, ridx_ref, ridx_col_ref, ew_ref, out_ref,
             ew_full, ridx_full, send_ew, recv_ew, send_ri, recv_ri):
        me = lax.axis_index("i")

        ridx_full[pl.ds(me, 1), :] = ridx_ref[...]

        sends = []
        for d in range(1, N_DEV):
            tgt = lax.rem(me + d, N_DEV)
            r_ew = pltpu.make_async_remote_copy(
                src_ref=ew_ref,
                dst_ref=ew_full.at[me],
                send_sem=send_ew.at[d - 1],
                recv_sem=recv_ew.at[me],
                device_id=(tgt,),
                device_id_type=pl.DeviceIdType.MESH,
            )
            r_ew.start()
            r_ri = pltpu.make_async_remote_copy(
                src_ref=ridx_ref,
                dst_ref=ridx_full.at[pl.ds(me, 1), :],
                send_sem=send_ri.at[d - 1],
                recv_sem=recv_ri.at[me],
                device_id=(tgt,),
                device_id_type=pl.DeviceIdType.MESH,
            )
            r_ri.start()
            sends.append(r_ew)
            sends.append(r_ri)

        for d in range(1, N_DEV):
            src = lax.rem(me - d + N_DEV, N_DEV)
            pltpu.make_async_remote_copy(
                src_ref=ridx_ref,
                dst_ref=ridx_full.at[pl.ds(src, 1), :],
                send_sem=send_ri.at[0],
                recv_sem=recv_ri.at[src],
                device_id=(src,),
                device_id_type=pl.DeviceIdType.MESH,
            ).wait_recv()

        ridx_all = ridx_full[...]
        iota_e = lax.broadcasted_iota(
            jnp.int32, (N_DEV, T_LOC, N_EXP), 2)
        onehot3 = (ridx_all[:, :, None] == iota_e).astype(jnp.float32)
        shard_iota = lax.broadcasted_iota(jnp.int32, (N_DEV, 1, 1), 0)
        before = (shard_iota < me).astype(jnp.float32)
        base = jnp.sum(onehot3 * before, axis=(0, 1))
        e_col = ridx_col_ref[...]
        iota_lane = lax.broadcasted_iota(jnp.int32, (T_LOC, N_EXP), 1)
        onehot_loc = (e_col == iota_lane).astype(jnp.float32)
        ii = lax.broadcasted_iota(jnp.int32, (T_LOC, T_LOC), 0)
        jj = lax.broadcasted_iota(jnp.int32, (T_LOC, T_LOC), 1)
        ltri = (jj < ii).astype(jnp.float32)
        excl = jnp.dot(ltri, onehot_loc,
                       preferred_element_type=jnp.float32)
        rank = jnp.sum(onehot_loc * (base[None, :] + excl),
                       axis=1, keepdims=True)
        keep = (rank < CAP).astype(jnp.bfloat16)

        xk = x_ref[...] * keep

        def expert_pair(src, w_pair, acc):
            for k in range(E_PER):
                m = (e_col == E_PER * src + k).astype(jnp.bfloat16)
                acc = acc + jnp.dot(xk * m, w_pair[k],
                                    preferred_element_type=jnp.float32)
            return acc

        acc = jnp.zeros((T_LOC, H), jnp.float32)
        acc = expert_pair(me, ew_ref[...], acc)
        for d in range(1, N_DEV):
            src = lax.rem(me - d + N_DEV, N_DEV)
            pltpu.make_async_remote_copy(
                src_ref=ew_ref,
                dst_ref=ew_full.at[src],
                send_sem=send_ew.at[0],
                recv_sem=recv_ew.at[src],
                device_id=(src,),
                device_id_type=pl.DeviceIdType.MESH,
            ).wait_recv()
            acc = expert_pair(src, ew_full[pl.ds(src, 1)][0], acc)
        out_ref[...] = acc

        for r in sends:
            r.wait_send()

    return pl.pallas_call(
        body,
        out_shape=jax.ShapeDtypeStruct((T_LOC, H), jnp.float32),
        in_specs=[pl.BlockSpec(memory_space=pltpu.VMEM)] * 4,
        out_specs=pl.BlockSpec(memory_space=pltpu.VMEM),
        scratch_shapes=[
            pltpu.VMEM((N_DEV, E_PER, D, H), jnp.bfloat16),
            pltpu.VMEM((N_DEV, T_LOC), jnp.int32),
            pltpu.SemaphoreType.DMA((N_DEV - 1,)),
            pltpu.SemaphoreType.DMA((N_DEV,)),
            pltpu.SemaphoreType.DMA((N_DEV - 1,)),
            pltpu.SemaphoreType.DMA((N_DEV,)),
        ],
    )(x, ridx_row, ridx_col, ew)


# device time: 70035 ns/iter; 1.0663x vs baseline; 1.0663x over previous
import jax
import jax.numpy as jnp
from jax import lax
from jax.experimental import pallas as pl
from jax.experimental.pallas import tpu as pltpu

N_DEV = 32
E_PER = 2
N_EXP = 64
T_LOC = 256
D = 128
H = 256
CAP = 102


def kernel(x, router_W, route_idx, expert_W):
    del router_W
    x = x.astype(jnp.bfloat16)
    ew = expert_W.astype(jnp.bfloat16)
    ridx_row = route_idx.reshape(1, T_LOC)
    ridx_col = route_idx.reshape(T_LOC, 1)

    def body(x_ref, ridx_ref, ridx_col_ref, ew_ref, out_ref,
             ew_full, ridx_full, send_ew, recv_ew, send_ri, recv_ri):
        me = lax.axis_index("i")

        ridx_full[pl.ds(me, 1), :] = ridx_ref[...]

        sends = []
        for d in range(1, N_DEV):
            tgt = lax.rem(me + d, N_DEV)
            r_ew = pltpu.make_async_remote_copy(
                src_ref=ew_ref,
                dst_ref=ew_full.at[me],
                send_sem=send_ew.at[d - 1],
                recv_sem=recv_ew.at[me],
                device_id=(tgt,),
                device_id_type=pl.DeviceIdType.MESH,
            )
            r_ew.start()
            r_ri = pltpu.make_async_remote_copy(
                src_ref=ridx_ref,
                dst_ref=ridx_full.at[pl.ds(me, 1), :],
                send_sem=send_ri.at[d - 1],
                recv_sem=recv_ri.at[me],
                device_id=(tgt,),
                device_id_type=pl.DeviceIdType.MESH,
            )
            r_ri.start()
            sends.append(r_ew)
            sends.append(r_ri)

        for d in range(1, N_DEV):
            src = lax.rem(me - d + N_DEV, N_DEV)
            pltpu.make_async_remote_copy(
                src_ref=ridx_ref,
                dst_ref=ridx_full.at[pl.ds(src, 1), :],
                send_sem=send_ri.at[0],
                recv_sem=recv_ri.at[src],
                device_id=(src,),
                device_id_type=pl.DeviceIdType.MESH,
            ).wait_recv()

        ridx_all = ridx_full[...]
        iota_e = lax.broadcasted_iota(
            jnp.int32, (N_DEV, T_LOC, N_EXP), 2)
        onehot3 = (ridx_all[:, :, None] == iota_e).astype(jnp.float32)
        shard_iota = lax.broadcasted_iota(jnp.int32, (N_DEV, 1, 1), 0)
        before = (shard_iota < me).astype(jnp.float32)
        base = jnp.sum(onehot3 * before, axis=(0, 1))
        e_col = ridx_col_ref[...]
        iota_lane = lax.broadcasted_iota(jnp.int32, (T_LOC, N_EXP), 1)
        onehot_loc = (e_col == iota_lane).astype(jnp.float32)
        ii = lax.broadcasted_iota(jnp.int32, (T_LOC, T_LOC), 0)
        jj = lax.broadcasted_iota(jnp.int32, (T_LOC, T_LOC), 1)
        ltri = (jj < ii).astype(jnp.float32)
        excl = jnp.dot(ltri, onehot_loc,
                       preferred_element_type=jnp.float32)
        rank = jnp.sum(onehot_loc * (base[None, :] + excl),
                       axis=1, keepdims=True)
        keep = (rank < CAP).astype(jnp.bfloat16)

        xk = x_ref[...] * keep

        ew_full[pl.ds(me, 1)] = ew_ref[...][None]
        for d in range(1, N_DEV):
            src = lax.rem(me - d + N_DEV, N_DEV)
            pltpu.make_async_remote_copy(
                src_ref=ew_ref,
                dst_ref=ew_full.at[src],
                send_sem=send_ew.at[0],
                recv_sem=recv_ew.at[src],
                device_id=(src,),
                device_id_type=pl.DeviceIdType.MESH,
            ).wait_recv()

        acc = jnp.zeros((T_LOC, H), jnp.float32)
        for e in range(N_EXP):
            m = onehot_loc[:, e:e + 1].astype(jnp.bfloat16)
            acc = acc + jnp.dot(xk * m, ew_full[e // E_PER, e % E_PER],
                                preferred_element_type=jnp.float32)
        out_ref[...] = acc

        for r in sends:
            r.wait_send()

    return pl.pallas_call(
        body,
        out_shape=jax.ShapeDtypeStruct((T_LOC, H), jnp.float32),
        in_specs=[pl.BlockSpec(memory_space=pltpu.VMEM)] * 4,
        out_specs=pl.BlockSpec(memory_space=pltpu.VMEM),
        scratch_shapes=[
            pltpu.VMEM((N_DEV, E_PER, D, H), jnp.bfloat16),
            pltpu.VMEM((N_DEV, T_LOC), jnp.int32),
            pltpu.SemaphoreType.DMA((N_DEV - 1,)),
            pltpu.SemaphoreType.DMA((N_DEV,)),
            pltpu.SemaphoreType.DMA((N_DEV - 1,)),
            pltpu.SemaphoreType.DMA((N_DEV,)),
        ],
    )(x, ridx_row, ridx_col, ew)
